# baseline (speedup 1.0000x reference)
import functools

import jax
import jax.numpy as jnp
from jax import lax
from jax.experimental import pallas as pl
from jax.experimental.pallas import tpu as pltpu
from jax.experimental.pallas import tpu_sc as plsc

S, D, F, E, K = 2048, 1024, 4096, 8, 2
NA = S * K
BM = 256
NB = (NA + E * (BM - 1) + BM - 1) // BM
NR = NB * BM
FSPLIT = 2
FB = F // FSPLIT

_NC, _NS = 2, 16
_NW = _NC * _NS
_RPW = NR // _NW
_GCH = 64
_TPW = S // _NW
_TCH = 32


def _router_body(x_ref, wr_ref, br_ref, idx_ref, w_ref):
    x = x_ref[...]
    logits = lax.dot_general(x, wr_ref[...], (((1,), (0,)), ((), ())),
                             preferred_element_type=jnp.float32)
    logits = logits + br_ref[...]
    ids = lax.broadcasted_iota(jnp.int32, (S, E), 1)
    v1 = jnp.max(logits, axis=1, keepdims=True)
    i1 = jnp.min(jnp.where(logits == v1, ids, E), axis=1, keepdims=True)
    neg = jnp.float32(-3.4e38)
    l2 = jnp.where(ids == i1, neg, logits)
    v2 = jnp.max(l2, axis=1, keepdims=True)
    i2 = jnp.min(jnp.where(l2 == v2, ids, E), axis=1, keepdims=True)
    t = jnp.exp(v2 - v1)
    w1 = 1.0 / (1.0 + t)
    w2 = t / (1.0 + t)
    lane = lax.broadcasted_iota(jnp.int32, (S, K), 1)
    idx_ref[...] = jnp.where(lane == 0, i1, i2)
    w_ref[...] = jnp.where(lane == 0, w1, w2)


def _router(x, Wr, br2):
    return pl.pallas_call(
        _router_body,
        out_shape=(jax.ShapeDtypeStruct((S, K), jnp.int32),
                   jax.ShapeDtypeStruct((S, K), jnp.float32)),
    )(x, Wr, br2)


def _sc_gather(x, src):
    mesh = plsc.VectorSubcoreMesh(core_axis_name="c", subcore_axis_name="s")

    @functools.partial(
        pl.kernel,
        out_type=jax.ShapeDtypeStruct((NR, D), jnp.float32),
        mesh=mesh,
        scratch_types=[pltpu.VMEM((_GCH,), jnp.int32),
                       pltpu.VMEM((_GCH, D), jnp.float32),
                       pltpu.SemaphoreType.DMA],
    )
    def k(x_hbm, src_hbm, out_hbm, idx_v, rows_v, sem):
        wid = lax.axis_index("s") * _NC + lax.axis_index("c")
        base = wid * _RPW

        def chunk(c, carry):
            off = base + c * _GCH
            pltpu.sync_copy(src_hbm.at[pl.ds(off, _GCH)], idx_v)
            pltpu.async_copy(x_hbm.at[idx_v], rows_v, sem).wait()
            pltpu.sync_copy(rows_v, out_hbm.at[pl.ds(off, _GCH)])
            return carry

        lax.fori_loop(0, _RPW // _GCH, chunk, 0)

    return k(x, src)


def _ffn_body(s_ref, xs_ref, w1_ref, b1_ref, w2_ref, b2_ref, ws_ref, y_ref):
    b = pl.program_id(0)
    f = pl.program_id(1)

    @pl.when(b < s_ref[0])
    def _():
        xb = xs_ref[...]
        h = lax.dot_general(xb, w1_ref[0], (((1,), (0,)), ((), ())),
                            preferred_element_type=jnp.float32)
        h = jnp.maximum(h + b1_ref[0, 0], 0.0)
        y = lax.dot_general(h, w2_ref[0], (((1,), (0,)), ((), ())),
                            preferred_element_type=jnp.float32)

        @pl.when(f == 0)
        def _():
            y_ref[...] = y + b2_ref[0]

        @pl.when(f != 0)
        def _():
            y_ref[...] += y

        @pl.when(f == FSPLIT - 1)
        def _():
            y_ref[...] = y_ref[...] * ws_ref[0]


def _ffn(sp, xs, W1, b1, W2, b2, ws):
    grid_spec = pltpu.PrefetchScalarGridSpec(
        num_scalar_prefetch=1,
        grid=(NB, FSPLIT),
        in_specs=[
            pl.BlockSpec((BM, D), lambda b, f, s: (b, 0)),
            pl.BlockSpec((1, D, FB), lambda b, f, s: (s[1 + b], 0, f)),
            pl.BlockSpec((1, 1, 1, FB), lambda b, f, s: (s[1 + b], f, 0, 0)),
            pl.BlockSpec((1, FB, D), lambda b, f, s: (s[1 + b], f, 0)),
            pl.BlockSpec((1, 1, D), lambda b, f, s: (s[1 + b], 0, 0)),
            pl.BlockSpec((1, BM, 1), lambda b, f, s: (b, 0, 0)),
        ],
        out_specs=pl.BlockSpec((BM, D), lambda b, f, s: (b, 0)),
    )
    return pl.pallas_call(
        _ffn_body,
        grid_spec=grid_spec,
        out_shape=jax.ShapeDtypeStruct((NR, D), jnp.float32),
        compiler_params=pltpu.CompilerParams(
            dimension_semantics=("arbitrary", "arbitrary")),
    )(sp, xs, W1, b1.reshape(E, FSPLIT, 1, FB), W2, b2.reshape(E, 1, D),
      ws.reshape(NB, BM, 1))


def _sc_combine(y, posflat):
    mesh = plsc.VectorSubcoreMesh(core_axis_name="c", subcore_axis_name="s")

    @functools.partial(
        pl.kernel,
        out_type=jax.ShapeDtypeStruct((S, D), jnp.float32),
        mesh=mesh,
        scratch_types=[pltpu.VMEM((K * _TCH,), jnp.int32),
                       pltpu.VMEM((K * _TCH, D), jnp.float32),
                       pltpu.VMEM((_TCH, D), jnp.float32),
                       pltpu.SemaphoreType.DMA],
    )
    def k(y_hbm, pos_hbm, out_hbm, idx_v, rows_v, out_v, sem):
        wid = lax.axis_index("s") * _NC + lax.axis_index("c")
        tbase = wid * _TPW

        def chunk(c, carry):
            t0 = tbase + c * _TCH
            pltpu.sync_copy(pos_hbm.at[pl.ds(t0 * K, K * _TCH)], idx_v)
            pltpu.async_copy(y_hbm.at[idx_v], rows_v, sem).wait()

            def tok(i, carry2):
                def lanes(cc, carry3):
                    sl = pl.ds(cc * 16, 16)
                    out_v[i, sl] = (rows_v[2 * i, sl]
                                    + rows_v[2 * i + 1, sl])
                    return carry3

                lax.fori_loop(0, D // 16, lanes, 0)
                return carry2

            lax.fori_loop(0, _TCH, tok, 0)
            pltpu.sync_copy(out_v, out_hbm.at[pl.ds(t0, _TCH)])
            return carry

        lax.fori_loop(0, _TPW // _TCH, chunk, 0)

    return k(y, posflat)


def kernel(inputs, Wr, br, W1, b1, W2, b2):
    x = inputs.reshape(S, D)
    idx, w = _router(x, Wr, br.reshape(1, E))

    a = idx.reshape(-1)
    onehot = (a[:, None] == jnp.arange(E, dtype=jnp.int32)).astype(jnp.int32)
    csum = jnp.cumsum(onehot, axis=0)
    counts = csum[-1]
    rank = jnp.take_along_axis(csum, a[:, None], axis=1)[:, 0] - 1
    padded = ((counts + BM - 1) // BM) * BM
    pad_end = jnp.cumsum(padded)
    pad_off = pad_end - padded
    pos = (pad_off[a] + rank).astype(jnp.int32)
    nact = (pad_end[-1] // BM).astype(jnp.int32)
    src = jnp.zeros((NR,), jnp.int32).at[pos].set(
        jnp.arange(NA, dtype=jnp.int32) // K)
    be = jnp.searchsorted(
        pad_end, jnp.arange(NB, dtype=jnp.int32) * BM,
        side="right").astype(jnp.int32)
    be = jnp.minimum(be, be[nact - 1])
    sp = jnp.concatenate([nact[None], be])

    ws = jnp.zeros((NR,), jnp.float32).at[pos].set(w.reshape(-1))

    xs = _sc_gather(x, src)
    y = _ffn(sp, xs, W1, b1, W2, b2, ws)
    out = _sc_combine(y, pos)
    return out.reshape(1, S, D)
